# baseline (speedup 1.0000x reference)
import functools

import jax
import jax.numpy as jnp
from jax import lax
from jax.experimental import pallas as pl
from jax.experimental.pallas import tpu as pltpu
from jax.experimental.pallas import tpu_sc as plsc

_B, _KPT, _N, _D = 4, 256, 8192, 128
_K = 16
_TAU = 5.0
_R = _B * _KPT
_ROWS = _R * _K
_F32 = jnp.float32


def _dot(a, b):
    return jnp.dot(a, b, preferred_element_type=_F32)


def _bn_rows(h, g, b):
    m = jnp.mean(h, axis=0, keepdims=True)
    v = jnp.mean((h - m) ** 2, axis=0, keepdims=True)
    return (h - m) / jnp.sqrt(v + 1e-5) * g + b


def _knn_body(k3_ref, pt_ref, idx_ref):
    k3 = k3_ref[0]
    pt = pt_ref[0]
    score = jnp.zeros((k3.shape[0], _N), _F32)
    for d in range(3):
        t = k3[:, d:d + 1] - pt[d:d + 1, :]
        score = score + t * t
    qw = _N // 4
    li = lax.broadcasted_iota(jnp.int32, (k3.shape[0], qw), 1)
    vs = [score[:, q * qw:(q + 1) * qw] for q in range(4)]
    ix = [li + q * qw for q in range(4)]

    def ce(x, y):
        swap = vs[y] < vs[x]
        vs[x], vs[y] = (jnp.where(swap, vs[y], vs[x]),
                        jnp.where(swap, vs[x], vs[y]))
        ix[x], ix[y] = (jnp.where(swap, ix[y], ix[x]),
                        jnp.where(swap, ix[x], ix[y]))

    for x, y in ((0, 1), (2, 3), (0, 2), (1, 3), (1, 2)):
        ce(x, y)
    big = _F32(3.4e38)
    cols = []
    for _ in range(_K):
        m = jnp.min(vs[0], axis=1, keepdims=True)
        eq = vs[0] <= m
        j = jnp.min(jnp.where(eq, ix[0], _N), axis=1, keepdims=True)
        cols.append(j)
        vs[0] = jnp.where(eq, vs[1], vs[0])
        ix[0] = jnp.where(eq, ix[1], ix[0])
        vs[1] = jnp.where(eq, vs[2], vs[1])
        ix[1] = jnp.where(eq, ix[2], ix[1])
        vs[2] = jnp.where(eq, vs[3], vs[2])
        ix[2] = jnp.where(eq, ix[3], ix[2])
        vs[3] = jnp.where(eq, big, vs[3])
    idx_ref[0] = jnp.concatenate(cols, axis=1)


def _knn(k3d8, pts_t):
    return pl.pallas_call(
        _knn_body,
        grid=(_B,),
        in_specs=[pl.BlockSpec((1, _KPT, 8), lambda b: (b, 0, 0)),
                  pl.BlockSpec((1, 8, _N), lambda b: (b, 0, 0))],
        out_specs=pl.BlockSpec((1, _KPT, _K), lambda b: (b, 0, 0)),
        out_shape=jax.ShapeDtypeStruct((_B, _KPT, _K), jnp.int32),
    )(k3d8, pts_t)


def _sc_gather(gidx_flat, feat_tab, xyz_flat):
    info = plsc.get_sparse_core_info()
    nc, ns = info.num_cores, info.num_subcores
    nw = nc * ns
    rpw = _ROWS // nw
    nchunk = rpw // 128
    wpb = nw // _B
    mesh = plsc.VectorSubcoreMesh(core_axis_name="c", subcore_axis_name="s")

    @functools.partial(
        pl.kernel, mesh=mesh,
        out_type=(jax.ShapeDtypeStruct((_ROWS, _D), _F32),
                  jax.ShapeDtypeStruct((_ROWS, 16), _F32)),
        scratch_types=[pltpu.VMEM((nchunk, 128), jnp.int32),
                       pltpu.VMEM((rpw,), jnp.int32),
                       pltpu.VMEM((rpw // 2, _D), _F32),
                       pltpu.VMEM((3 * _N,), _F32),
                       pltpu.VMEM((rpw, 16), _F32),
                       pltpu.SemaphoreType.DMA],
        compiler_params=pltpu.CompilerParams(needs_layout_passes=False),
    )
    def gather_k(idx_hbm, idx2_hbm, feat_hbm, xyz_hbm, feat_out, xyz_out,
                 idx_v, idx_f, rows_v, xyz_loc, xo, sem):
        wid = lax.axis_index("s") * nc + lax.axis_index("c")
        base = wid * rpw
        b = wid // wpb
        half = rpw // 2
        hchunk = nchunk // 2
        pltpu.sync_copy(idx_hbm.at[wid], idx_v)
        handles = [pltpu.async_copy(
            feat_hbm.at[idx_v.at[j]], rows_v.at[pl.ds(j * 128, 128)], sem)
            for j in range(hchunk)]
        pltpu.sync_copy(idx2_hbm.at[wid], idx_f)
        pltpu.sync_copy(xyz_hbm.at[pl.ds(b * 3 * _N, 3 * _N)], xyz_loc)
        for h in handles:
            h.wait()
        pltpu.sync_copy(rows_v, feat_out.at[pl.ds(base, half)])
        handles = [pltpu.async_copy(
            feat_hbm.at[idx_v.at[hchunk + j]],
            rows_v.at[pl.ds(j * 128, 128)], sem)
            for j in range(hchunk)]
        lane = lax.broadcasted_iota(jnp.int32, (16,), 0)
        off = b * _N
        for g in range(rpw // 16):
            iv = idx_f[pl.ds(g * 16, 16)] - off
            p = iv * 3
            rows = lane + g * 16
            for d in range(3):
                vals = plsc.load_gather(xyz_loc, [p + d])
                plsc.store_scatter(xo, [rows, lane * 0 + d], vals)
        for h in handles:
            h.wait()
        pltpu.sync_copy(rows_v, feat_out.at[pl.ds(base + half, half)])
        pltpu.sync_copy(xo, xyz_out.at[pl.ds(base, rpw)])

    return gather_k(gidx_flat.reshape(nw, nchunk, 128),
                    gidx_flat.reshape(nw, rpw), feat_tab, xyz_flat)


def _q_body(kf_ref, w0, b0, g, bb, w1, b1, q_ref):
    h = _dot(kf_ref[...], w0[...]) + b0[...]
    h = jax.nn.relu(_bn_rows(h, g[...], bb[...]))
    q_ref[...] = _dot(h, w1[...]) + b1[...]


def _q_kernel(kf, w0, b0, g, bb, w1, b1):
    return pl.pallas_call(
        _q_body,
        out_shape=jax.ShapeDtypeStruct((_R, _D), _F32),
    )(kf, w0, b0, g, bb, w1, b1)


_NB_TILES = 4
_NB_KP = _KPT * _B // _NB_TILES
_NB_ROWS = _NB_KP * _K


def _nb_body(xyz_ref, feat_ref, k3_ref, q_ref,
             dW0, dB0, dW1, dB1, dW2, dB2, eW0, eB0, eW1, eB1,
             kf_out, num_out, nb_out):
    kx = k3_ref[...][:, :3]
    nx = xyz_ref[...][:, :3].reshape(_NB_KP, _K, 3)
    diff = (kx[:, None, :] - nx).reshape(_NB_ROWS, 3)
    h = jax.nn.relu(_dot(diff, dW0[...]) + dB0[...])
    h = jax.nn.relu(_dot(h, dW1[...]) + dB1[...])
    pos = _dot(h, dW2[...]) + dB2[...]
    h2 = jax.nn.relu(_dot(feat_ref[...], eW0[pl.ds(0, _D), :])
                     + _dot(pos, eW0[pl.ds(_D, _D), :]) + eB0[...])
    kfeat = _dot(h2, eW1[...]) + eB1[...]
    kf_out[...] = kfeat
    kf3 = kfeat.reshape(_NB_KP, _K, _D)
    q = q_ref[...]
    num_out[...] = jnp.sum(q[:, None, :] * kf3, axis=2)
    nb_out[...] = jnp.maximum(jnp.sqrt(jnp.sum(kf3 * kf3, axis=2)), 1e-8)


def _nb_kernel(knn_xyz, knn_feat_raw, k3d16, q, wts):
    full = pl.BlockSpec(None, lambda t: (0, 0))
    return pl.pallas_call(
        _nb_body,
        grid=(_NB_TILES,),
        in_specs=[pl.BlockSpec((_NB_ROWS, 16), lambda t: (t, 0)),
                  pl.BlockSpec((_NB_ROWS, _D), lambda t: (t, 0)),
                  pl.BlockSpec((_NB_KP, 16), lambda t: (t, 0)),
                  pl.BlockSpec((_NB_KP, _D), lambda t: (t, 0))]
                 + [full] * 10,
        out_specs=[pl.BlockSpec((_NB_ROWS, _D), lambda t: (t, 0)),
                   pl.BlockSpec((_NB_KP, _K), lambda t: (t, 0)),
                   pl.BlockSpec((_NB_KP, _K), lambda t: (t, 0))],
        out_shape=[jax.ShapeDtypeStruct((_ROWS, _D), _F32),
                   jax.ShapeDtypeStruct((_R, _K), _F32),
                   jax.ShapeDtypeStruct((_R, _K), _F32)],
    )(knn_xyz, knn_feat_raw, k3d16, q, *wts)


_PEL_TILE = 128


def _pel_body(k3i_ref, k3a_ref, W0, B0, W1, B1, W2, B2, out_ref):
    ai = _dot(k3i_ref[0], W0[...])
    aj = _dot(k3a_ref[0], W0[...])
    h1 = jax.nn.relu(ai[:, None, :] - aj[None, :, :] + B0[...])
    h1 = h1.reshape(_PEL_TILE * _KPT, 64)
    h2 = jax.nn.relu(_dot(h1, W1[...]) + B1[...])
    m2 = jnp.mean(h2.reshape(_PEL_TILE, _KPT, _D), axis=1)
    out_ref[0] = _dot(m2, W2[...]) + B2[...]


def _pel_kernel(k3d3, wts):
    full = pl.BlockSpec(None, lambda b, t: (0, 0))
    return pl.pallas_call(
        _pel_body,
        grid=(_B, _KPT // _PEL_TILE),
        in_specs=[pl.BlockSpec((1, _PEL_TILE, 16), lambda b, t: (b, t, 0)),
                  pl.BlockSpec((1, _KPT, 16), lambda b, t: (b, 0, 0))]
                 + [full] * 6,
        out_specs=pl.BlockSpec((1, _PEL_TILE, _D), lambda b, t: (b, t, 0)),
        out_shape=jax.ShapeDtypeStruct((_B, _KPT, _D), _F32),
    )(k3d3, k3d3, *wts)


def _af_body(kf_ref, q_ref, num_ref, nb_ref, feat_ref, pel_ref, k3_ref,
             aW0, aB0, aW1, aB1, aW2, aB2,
             fW0, fB0, fW1, fB1, g1, bb1, g2, bb2,
             oW0, oB0, oW1, oB1, out_ref):
    q = q_ref[...]
    na = jnp.maximum(jnp.sqrt(jnp.sum(q * q, axis=1, keepdims=True)), 1e-8)
    att = num_ref[...] / (na * nb_ref[...]) / _TAU
    att = att - jnp.max(att, axis=1, keepdims=True)
    e = jnp.exp(att)
    sim = e / jnp.sum(e, axis=1, keepdims=True)
    kf3 = feat_ref[...].reshape(_R, _K, _D)
    agg = jnp.sum(sim[:, :, None] * kf3, axis=1)
    kf1 = jax.nn.relu(agg + kf_ref[...])
    k3 = k3_ref[...]
    h = jax.nn.relu(_dot(k3, aW0[...]) + aB0[...])
    h = jax.nn.relu(_dot(h, aW1[...]) + aB1[...])
    pea = _dot(h, aW2[...]) + aB2[...]
    pos_l = pea + pel_ref[...]
    g = jnp.mean(kf1.reshape(_B, _KPT, _D), axis=1)
    hg = _dot(g, fW0[pl.ds(_D, _D), :])
    hgb = jnp.broadcast_to(hg[:, None, :], (_B, _KPT, _D)).reshape(_R, _D)
    h = (_dot(kf1, fW0[pl.ds(0, _D), :]) + hgb
         + _dot(pos_l, fW0[pl.ds(2 * _D, _D), :]) + fB0[...])
    h = jax.nn.relu(_bn_rows(h, g1[...], bb1[...]))
    h = _dot(h, fW1[...]) + fB1[...]
    h = jax.nn.relu(_bn_rows(h, g2[...], bb2[...]))
    kf2 = jax.nn.relu(h + kf1)
    o = _dot(jax.nn.relu(_dot(kf2, oW0[...]) + oB0[...]), oW1[...]) + oB1[...]
    out_ref[...] = jax.nn.relu(kf2 + o)


def _af_kernel(kf, q, num, nb, knn_feat, pel, k3d16, wts):
    return pl.pallas_call(
        _af_body,
        out_shape=jax.ShapeDtypeStruct((_R, _D), _F32),
    )(kf, q, num, nb, knn_feat, pel, k3d16, *wts)


def _pad_w3(w):
    return jnp.pad(w, ((0, 13), (0, 0)))


def _row(b):
    return b[None, :]


def kernel(kpt_feature, kpt_3d, pts_feature, pts, params):
    k3d8 = jnp.pad(kpt_3d, ((0, 0), (0, 0), (0, 5)))
    pts_t = jnp.pad(jnp.transpose(pts, (0, 2, 1)), ((0, 0), (0, 5), (0, 0)))
    idx = _knn(k3d8, pts_t)

    goff = (jnp.arange(_B, dtype=jnp.int32) * _N)[:, None, None]
    gidx = (idx + goff).reshape(_ROWS)
    feat_tab = pts_feature.reshape(_B * _N, _D)
    xyz_flat = pts.reshape(_B * _N * 3)
    knn_feat_raw, knn_xyz = _sc_gather(gidx, feat_tab, xyz_flat)

    k3d16_3 = jnp.pad(kpt_3d, ((0, 0), (0, 0), (0, 13)))
    k3d16 = k3d16_3.reshape(_R, 16)
    kf = kpt_feature.reshape(_R, _D)

    for p in params:
        q = _q_kernel(kf,
                      p["fc_in"][0]["W"], _row(p["fc_in"][0]["b"]),
                      _row(p["bn_in"]["g"]), _row(p["bn_in"]["b"]),
                      p["fc_in"][1]["W"], _row(p["fc_in"][1]["b"]))
        nb_wts = [p["fc_delta"][0]["W"], _row(p["fc_delta"][0]["b"]),
                  p["fc_delta"][1]["W"], _row(p["fc_delta"][1]["b"]),
                  p["fc_delta"][2]["W"], _row(p["fc_delta"][2]["b"]),
                  p["fc_delta_1"][0]["W"], _row(p["fc_delta_1"][0]["b"]),
                  p["fc_delta_1"][1]["W"], _row(p["fc_delta_1"][1]["b"])]
        knn_feat, num, nb = _nb_kernel(knn_xyz, knn_feat_raw, k3d16, q, nb_wts)
        pel_wts = [_pad_w3(p["fc_delta_l"][0]["W"]), _row(p["fc_delta_l"][0]["b"]),
                   p["fc_delta_l"][1]["W"], _row(p["fc_delta_l"][1]["b"]),
                   p["fc_delta_l"][2]["W"], _row(p["fc_delta_l"][2]["b"])]
        pel = _pel_kernel(k3d16_3, pel_wts).reshape(_R, _D)
        af_wts = [_pad_w3(p["fc_delta_abs"][0]["W"]), _row(p["fc_delta_abs"][0]["b"]),
                  p["fc_delta_abs"][1]["W"], _row(p["fc_delta_abs"][1]["b"]),
                  p["fc_delta_abs"][2]["W"], _row(p["fc_delta_abs"][2]["b"]),
                  p["fuse"][0]["W"], _row(p["fuse"][0]["b"]),
                  p["fuse"][1]["W"], _row(p["fuse"][1]["b"]),
                  _row(p["bn_f1"]["g"]), _row(p["bn_f1"]["b"]),
                  _row(p["bn_f2"]["g"]), _row(p["bn_f2"]["b"]),
                  p["out_mlp"][0]["W"], _row(p["out_mlp"][0]["b"]),
                  p["out_mlp"][1]["W"], _row(p["out_mlp"][1]["b"])]
        kf = _af_kernel(kf, q, num, nb, knn_feat, pel, k3d16, af_wts)

    return kf.reshape(_B, _KPT, _D)


# speedup vs baseline: 1.0268x; 1.0268x over previous
import functools

import jax
import jax.numpy as jnp
from jax import lax
from jax.experimental import pallas as pl
from jax.experimental.pallas import tpu as pltpu
from jax.experimental.pallas import tpu_sc as plsc

_B, _KPT, _N, _D = 4, 256, 8192, 128
_K = 16
_TAU = 5.0
_R = _B * _KPT
_ROWS = _R * _K
_F32 = jnp.float32


def _dot(a, b):
    return jnp.dot(a, b, preferred_element_type=_F32)


def _bn_rows(h, g, b):
    m = jnp.mean(h, axis=0, keepdims=True)
    v = jnp.mean((h - m) ** 2, axis=0, keepdims=True)
    return (h - m) / jnp.sqrt(v + 1e-5) * g + b


def _knn_body(k3_ref, pt_ref, idx_ref):
    k3 = k3_ref[0]
    pt = pt_ref[0]
    score = jnp.zeros((k3.shape[0], _N), _F32)
    for d in range(3):
        t = k3[:, d:d + 1] - pt[d:d + 1, :]
        score = score + t * t
    qw = _N // 4
    li = lax.broadcasted_iota(jnp.int32, (k3.shape[0], qw), 1)
    vs = [score[:, q * qw:(q + 1) * qw] for q in range(4)]
    ix = [li + q * qw for q in range(4)]

    def ce(x, y):
        swap = vs[y] < vs[x]
        vs[x], vs[y] = (jnp.where(swap, vs[y], vs[x]),
                        jnp.where(swap, vs[x], vs[y]))
        ix[x], ix[y] = (jnp.where(swap, ix[y], ix[x]),
                        jnp.where(swap, ix[x], ix[y]))

    for x, y in ((0, 1), (2, 3), (0, 2), (1, 3), (1, 2)):
        ce(x, y)
    big = _F32(3.4e38)
    cols = []
    for it in range(_K):
        m = jnp.min(vs[0], axis=1, keepdims=True)
        eq = vs[0] <= m
        j = jnp.min(jnp.where(eq, ix[0], _N), axis=1, keepdims=True)
        cols.append(j)
        if it + 1 < _K:
            vs[0] = jnp.where(eq, vs[1], vs[0])
            ix[0] = jnp.where(eq, ix[1], ix[0])
            vs[1] = jnp.where(eq, vs[2], vs[1])
            ix[1] = jnp.where(eq, ix[2], ix[1])
            vs[2] = jnp.where(eq, vs[3], vs[2])
            ix[2] = jnp.where(eq, ix[3], ix[2])
            vs[3] = jnp.where(eq, big, vs[3])
    idx_ref[0] = jnp.concatenate(cols, axis=1)


def _knn(k3d8, pts_t):
    return pl.pallas_call(
        _knn_body,
        grid=(_B, 2),
        in_specs=[pl.BlockSpec((1, 128, 8), lambda b, t: (b, t, 0)),
                  pl.BlockSpec((1, 8, _N), lambda b, t: (b, 0, 0))],
        out_specs=pl.BlockSpec((1, 128, _K), lambda b, t: (b, t, 0)),
        out_shape=jax.ShapeDtypeStruct((_B, _KPT, _K), jnp.int32),
    )(k3d8, pts_t)


def _sc_gather(gidx_flat, feat_tab, xyz_flat):
    info = plsc.get_sparse_core_info()
    nc, ns = info.num_cores, info.num_subcores
    nw = nc * ns
    rpw = _ROWS // nw
    nchunk = rpw // 128
    wpb = nw // _B
    mesh = plsc.VectorSubcoreMesh(core_axis_name="c", subcore_axis_name="s")

    @functools.partial(
        pl.kernel, mesh=mesh,
        out_type=(jax.ShapeDtypeStruct((_ROWS, _D), _F32),
                  jax.ShapeDtypeStruct((_ROWS, 16), _F32)),
        scratch_types=[pltpu.VMEM((nchunk, 128), jnp.int32),
                       pltpu.VMEM((rpw,), jnp.int32),
                       pltpu.VMEM((rpw // 2, _D), _F32),
                       pltpu.VMEM((3 * _N,), _F32),
                       pltpu.VMEM((rpw, 16), _F32),
                       pltpu.SemaphoreType.DMA],
        compiler_params=pltpu.CompilerParams(needs_layout_passes=False),
    )
    def gather_k(idx_hbm, idx2_hbm, feat_hbm, xyz_hbm, feat_out, xyz_out,
                 idx_v, idx_f, rows_v, xyz_loc, xo, sem):
        wid = lax.axis_index("s") * nc + lax.axis_index("c")
        base = wid * rpw
        b = wid // wpb
        half = rpw // 2
        hchunk = nchunk // 2
        pltpu.sync_copy(idx_hbm.at[wid], idx_v)
        handles = [pltpu.async_copy(
            feat_hbm.at[idx_v.at[j]], rows_v.at[pl.ds(j * 128, 128)], sem)
            for j in range(hchunk)]
        pltpu.sync_copy(idx2_hbm.at[wid], idx_f)
        pltpu.sync_copy(xyz_hbm.at[pl.ds(b * 3 * _N, 3 * _N)], xyz_loc)
        for h in handles:
            h.wait()
        pltpu.sync_copy(rows_v, feat_out.at[pl.ds(base, half)])
        handles = [pltpu.async_copy(
            feat_hbm.at[idx_v.at[hchunk + j]],
            rows_v.at[pl.ds(j * 128, 128)], sem)
            for j in range(hchunk)]
        lane = lax.broadcasted_iota(jnp.int32, (16,), 0)
        off = b * _N
        for g in range(rpw // 16):
            iv = idx_f[pl.ds(g * 16, 16)] - off
            p = iv * 3
            rows = lane + g * 16
            for d in range(3):
                vals = plsc.load_gather(xyz_loc, [p + d])
                plsc.store_scatter(xo, [rows, lane * 0 + d], vals)
        for h in handles:
            h.wait()
        pltpu.sync_copy(rows_v, feat_out.at[pl.ds(base + half, half)])
        pltpu.sync_copy(xo, xyz_out.at[pl.ds(base, rpw)])

    return gather_k(gidx_flat.reshape(nw, nchunk, 128),
                    gidx_flat.reshape(nw, rpw), feat_tab, xyz_flat)


def _q_body(kf_ref, w0, b0, g, bb, w1, b1, q_ref):
    h = _dot(kf_ref[...], w0[...]) + b0[...]
    h = jax.nn.relu(_bn_rows(h, g[...], bb[...]))
    q_ref[...] = _dot(h, w1[...]) + b1[...]


def _q_kernel(kf, w0, b0, g, bb, w1, b1):
    return pl.pallas_call(
        _q_body,
        out_shape=jax.ShapeDtypeStruct((_R, _D), _F32),
    )(kf, w0, b0, g, bb, w1, b1)


_NB_TILES = 8
_NB_KP = _KPT * _B // _NB_TILES
_NB_ROWS = _NB_KP * _K


def _nb_body(xyz_ref, feat_ref, k3_ref, q_ref,
             dW0, dB0, dW1, dB1, dW2, dB2, eW0, eB0, eW1, eB1,
             kf_out, num_out, nb_out):
    kx = k3_ref[...][:, :3]
    nx = xyz_ref[...][:, :3].reshape(_NB_KP, _K, 3)
    diff = (kx[:, None, :] - nx).reshape(_NB_ROWS, 3)
    h = jax.nn.relu(_dot(diff, dW0[...]) + dB0[...])
    h = jax.nn.relu(_dot(h, dW1[...]) + dB1[...])
    pos = _dot(h, dW2[...]) + dB2[...]
    h2 = jax.nn.relu(_dot(feat_ref[...], eW0[pl.ds(0, _D), :])
                     + _dot(pos, eW0[pl.ds(_D, _D), :]) + eB0[...])
    kfeat = _dot(h2, eW1[...]) + eB1[...]
    kf_out[...] = kfeat
    kf3 = kfeat.reshape(_NB_KP, _K, _D)
    q = q_ref[...]
    num_out[...] = jnp.sum(q[:, None, :] * kf3, axis=2)
    nb_out[...] = jnp.maximum(jnp.sqrt(jnp.sum(kf3 * kf3, axis=2)), 1e-8)


def _nb_kernel(knn_xyz, knn_feat_raw, k3d16, q, wts):
    full = pl.BlockSpec(None, lambda t: (0, 0))
    return pl.pallas_call(
        _nb_body,
        grid=(_NB_TILES,),
        in_specs=[pl.BlockSpec((_NB_ROWS, 16), lambda t: (t, 0)),
                  pl.BlockSpec((_NB_ROWS, _D), lambda t: (t, 0)),
                  pl.BlockSpec((_NB_KP, 16), lambda t: (t, 0)),
                  pl.BlockSpec((_NB_KP, _D), lambda t: (t, 0))]
                 + [full] * 10,
        out_specs=[pl.BlockSpec((_NB_ROWS, _D), lambda t: (t, 0)),
                   pl.BlockSpec((_NB_KP, _K), lambda t: (t, 0)),
                   pl.BlockSpec((_NB_KP, _K), lambda t: (t, 0))],
        out_shape=[jax.ShapeDtypeStruct((_ROWS, _D), _F32),
                   jax.ShapeDtypeStruct((_R, _K), _F32),
                   jax.ShapeDtypeStruct((_R, _K), _F32)],
    )(knn_xyz, knn_feat_raw, k3d16, q, *wts)


_PEL_TILE = 64


def _pel_body(k3i_ref, k3a_ref, W0, B0, W1, B1, W2, B2, out_ref):
    ai = _dot(k3i_ref[0], W0[...])
    aj = _dot(k3a_ref[0], W0[...])
    h1 = jax.nn.relu(ai[:, None, :] - aj[None, :, :] + B0[...])
    h1 = h1.reshape(_PEL_TILE * _KPT, 64)
    h2 = jax.nn.relu(_dot(h1, W1[...]) + B1[...])
    m2 = jnp.mean(h2.reshape(_PEL_TILE, _KPT, _D), axis=1)
    out_ref[0] = _dot(m2, W2[...]) + B2[...]


def _pel_kernel(k3d3, wts):
    full = pl.BlockSpec(None, lambda b, t: (0, 0))
    return pl.pallas_call(
        _pel_body,
        grid=(_B, _KPT // _PEL_TILE),
        in_specs=[pl.BlockSpec((1, _PEL_TILE, 16), lambda b, t: (b, t, 0)),
                  pl.BlockSpec((1, _KPT, 16), lambda b, t: (b, 0, 0))]
                 + [full] * 6,
        out_specs=pl.BlockSpec((1, _PEL_TILE, _D), lambda b, t: (b, t, 0)),
        out_shape=jax.ShapeDtypeStruct((_B, _KPT, _D), _F32),
    )(k3d3, k3d3, *wts)


def _af_core(kf_ref, q_ref, num_ref, nb_ref, feat_ref, pel_ref, k3_ref,
             aW0, aB0, aW1, aB1, aW2, aB2,
             fW0, fB0, fW1, fB1, g1, bb1, g2, bb2,
             oW0, oB0, oW1, oB1):
    q = q_ref[...]
    na = jnp.maximum(jnp.sqrt(jnp.sum(q * q, axis=1, keepdims=True)), 1e-8)
    att = num_ref[...] / (na * nb_ref[...]) / _TAU
    att = att - jnp.max(att, axis=1, keepdims=True)
    e = jnp.exp(att)
    sim = e / jnp.sum(e, axis=1, keepdims=True)
    kf3 = feat_ref[...].reshape(_R, _K, _D)
    agg = jnp.sum(sim[:, :, None] * kf3, axis=1)
    kf1 = jax.nn.relu(agg + kf_ref[...])
    k3 = k3_ref[...]
    h = jax.nn.relu(_dot(k3, aW0[...]) + aB0[...])
    h = jax.nn.relu(_dot(h, aW1[...]) + aB1[...])
    pea = _dot(h, aW2[...]) + aB2[...]
    pos_l = pea + pel_ref[...]
    g = jnp.mean(kf1.reshape(_B, _KPT, _D), axis=1)
    hg = _dot(g, fW0[pl.ds(_D, _D), :])
    hgb = jnp.broadcast_to(hg[:, None, :], (_B, _KPT, _D)).reshape(_R, _D)
    h = (_dot(kf1, fW0[pl.ds(0, _D), :]) + hgb
         + _dot(pos_l, fW0[pl.ds(2 * _D, _D), :]) + fB0[...])
    h = jax.nn.relu(_bn_rows(h, g1[...], bb1[...]))
    h = _dot(h, fW1[...]) + fB1[...]
    h = jax.nn.relu(_bn_rows(h, g2[...], bb2[...]))
    kf2 = jax.nn.relu(h + kf1)
    o = _dot(jax.nn.relu(_dot(kf2, oW0[...]) + oB0[...]), oW1[...]) + oB1[...]
    return jax.nn.relu(kf2 + o)


def _af_body(*args):
    out_ref = args[-1]
    out_ref[...] = _af_core(*args[:-1])


def _afq_body(*args):
    (nw0, nb0, ng, nbb, nw1, nb1, out_ref, q_ref) = args[-8:]
    out = _af_core(*args[:-8])
    out_ref[...] = out
    h = _dot(out, nw0[...]) + nb0[...]
    h = jax.nn.relu(_bn_rows(h, ng[...], nbb[...]))
    q_ref[...] = _dot(h, nw1[...]) + nb1[...]


def _af_kernel(kf, q, num, nb, knn_feat, pel, k3d16, wts):
    return pl.pallas_call(
        _af_body,
        out_shape=jax.ShapeDtypeStruct((_R, _D), _F32),
    )(kf, q, num, nb, knn_feat, pel, k3d16, *wts)


def _afq_kernel(kf, q, num, nb, knn_feat, pel, k3d16, wts, qwts):
    return pl.pallas_call(
        _afq_body,
        out_shape=(jax.ShapeDtypeStruct((_R, _D), _F32),
                   jax.ShapeDtypeStruct((_R, _D), _F32)),
    )(kf, q, num, nb, knn_feat, pel, k3d16, *wts, *qwts)


def _pad_w3(w):
    return jnp.pad(w, ((0, 13), (0, 0)))


def _row(b):
    return b[None, :]


def kernel(kpt_feature, kpt_3d, pts_feature, pts, params):
    k3d8 = jnp.pad(kpt_3d, ((0, 0), (0, 0), (0, 5)))
    pts_t = jnp.pad(jnp.transpose(pts, (0, 2, 1)), ((0, 0), (0, 5), (0, 0)))
    idx = _knn(k3d8, pts_t)

    goff = (jnp.arange(_B, dtype=jnp.int32) * _N)[:, None, None]
    gidx = (idx + goff).reshape(_ROWS)
    feat_tab = pts_feature.reshape(_B * _N, _D)
    xyz_flat = pts.reshape(_B * _N * 3)
    knn_feat_raw, knn_xyz = _sc_gather(gidx, feat_tab, xyz_flat)

    k3d16_3 = jnp.pad(kpt_3d, ((0, 0), (0, 0), (0, 13)))
    k3d16 = k3d16_3.reshape(_R, 16)
    kf = kpt_feature.reshape(_R, _D)

    def qwts(p):
        return [p["fc_in"][0]["W"], _row(p["fc_in"][0]["b"]),
                _row(p["bn_in"]["g"]), _row(p["bn_in"]["b"]),
                p["fc_in"][1]["W"], _row(p["fc_in"][1]["b"])]

    q = _q_kernel(kf, *qwts(params[0]))
    for i, p in enumerate(params):
        nb_wts = [p["fc_delta"][0]["W"], _row(p["fc_delta"][0]["b"]),
                  p["fc_delta"][1]["W"], _row(p["fc_delta"][1]["b"]),
                  p["fc_delta"][2]["W"], _row(p["fc_delta"][2]["b"]),
                  p["fc_delta_1"][0]["W"], _row(p["fc_delta_1"][0]["b"]),
                  p["fc_delta_1"][1]["W"], _row(p["fc_delta_1"][1]["b"])]
        knn_feat, num, nb = _nb_kernel(knn_xyz, knn_feat_raw, k3d16, q, nb_wts)
        pel_wts = [_pad_w3(p["fc_delta_l"][0]["W"]), _row(p["fc_delta_l"][0]["b"]),
                   p["fc_delta_l"][1]["W"], _row(p["fc_delta_l"][1]["b"]),
                   p["fc_delta_l"][2]["W"], _row(p["fc_delta_l"][2]["b"])]
        pel = _pel_kernel(k3d16_3, pel_wts).reshape(_R, _D)
        af_wts = [_pad_w3(p["fc_delta_abs"][0]["W"]), _row(p["fc_delta_abs"][0]["b"]),
                  p["fc_delta_abs"][1]["W"], _row(p["fc_delta_abs"][1]["b"]),
                  p["fc_delta_abs"][2]["W"], _row(p["fc_delta_abs"][2]["b"]),
                  p["fuse"][0]["W"], _row(p["fuse"][0]["b"]),
                  p["fuse"][1]["W"], _row(p["fuse"][1]["b"]),
                  _row(p["bn_f1"]["g"]), _row(p["bn_f1"]["b"]),
                  _row(p["bn_f2"]["g"]), _row(p["bn_f2"]["b"]),
                  p["out_mlp"][0]["W"], _row(p["out_mlp"][0]["b"]),
                  p["out_mlp"][1]["W"], _row(p["out_mlp"][1]["b"])]
        if i + 1 < len(params):
            kf, q = _afq_kernel(kf, q, num, nb, knn_feat, pel, k3d16,
                                af_wts, qwts(params[i + 1]))
        else:
            kf = _af_kernel(kf, q, num, nb, knn_feat, pel, k3d16, af_wts)

    return kf.reshape(_B, _KPT, _D)
